# baseline (speedup 1.0000x reference)
import functools

import jax
import jax.numpy as jnp
from jax import lax
from jax.experimental import pallas as pl
from jax.experimental.pallas import tpu as pltpu
from jax.experimental.pallas import tpu_sc as plsc

_CHUNK = 128
_NBUF = 4


def _lookup(idx, table):
    (n_rows,) = idx.shape
    v, d = table.shape
    chunk = _CHUNK
    info = plsc.get_sparse_core_info()
    nw = info.num_cores * info.num_subcores
    b_per_w = n_rows // nw
    n_chunks = b_per_w // chunk
    nbuf = min(_NBUF, n_chunks)

    mesh = plsc.VectorSubcoreMesh(core_axis_name="c", subcore_axis_name="s")

    @functools.partial(
        pl.kernel,
        mesh=mesh,
        out_type=jax.ShapeDtypeStruct((n_rows, d), jnp.float32),
        scratch_types=(
            [pltpu.VMEM_SHARED((v, d), jnp.float32)]
            + [pltpu.VMEM((b_per_w,), jnp.int32)]
            + [pltpu.VMEM((nbuf * chunk, d), jnp.float32)]
            + [pltpu.SemaphoreType.DMA((nbuf,))]
            + [pltpu.SemaphoreType.DMA((nbuf,))]
        ),
    )
    def k(table_hbm, idx_hbm, out_hbm, table_sh, idx_v, buf, gsems, ssems):
        sid = lax.axis_index("s")
        wid = sid * info.num_cores + lax.axis_index("c")
        @pl.when(sid == 0)
        def _():
            pltpu.sync_copy(table_hbm, table_sh)

        pltpu.sync_copy(idx_hbm.at[pl.ds(wid * b_per_w, b_per_w)], idx_v)
        plsc.subcore_barrier()

        out_base = wid * b_per_w

        def start_gather(c, slot, b):
            pltpu.async_copy(
                table_sh.at[idx_v.at[pl.ds(c * chunk, chunk)]],
                buf.at[pl.ds(slot, chunk)],
                gsems.at[b],
            )

        for b in range(nbuf):
            start_gather(b, b * chunk, b)

        def chunk_body(c, _):
            b = lax.rem(c, nbuf)
            slot = b * chunk
            bufslot = buf.at[pl.ds(slot, chunk)]
            pltpu.make_async_copy(
                out_hbm.at[pl.ds(out_base, chunk)], bufslot, gsems.at[b]
            ).wait()
            pltpu.async_copy(
                bufslot,
                out_hbm.at[pl.ds(out_base + c * chunk, chunk)],
                ssems.at[b],
            )

            @pl.when(c < n_chunks - nbuf)
            def _():
                pltpu.make_async_copy(
                    bufslot, out_hbm.at[pl.ds(out_base, chunk)], ssems.at[b]
                ).wait()
                start_gather(c + nbuf, slot, b)

            return 0

        lax.fori_loop(0, n_chunks, chunk_body, 0)
        for b in range(nbuf):
            pltpu.make_async_copy(
                buf.at[pl.ds(b * chunk, chunk)],
                out_hbm.at[pl.ds(out_base, chunk)],
                ssems.at[b],
            ).wait()

    return k(table, idx)


_TC_BLK = 2048


def _tc_lookup(idx, table):
    (n,) = idx.shape
    v, d = table.shape
    nb = n // _TC_BLK
    idx3 = idx.reshape(nb, 1, _TC_BLK)

    def body(idx_ref, table_ref, out_ref):
        idxb = idx_ref[0, 0, :]
        iota = lax.broadcasted_iota(jnp.int32, (_TC_BLK, v), 1)
        oh = (idxb[:, None] == iota).astype(jnp.float32)
        out_ref[...] = jnp.dot(
            oh, table_ref[...], preferred_element_type=jnp.float32
        )

    return pl.pallas_call(
        body,
        grid=(nb,),
        in_specs=[
            pl.BlockSpec((1, 1, _TC_BLK), lambda i: (i, 0, 0)),
            pl.BlockSpec((v, d), lambda i: (0, 0)),
        ],
        out_specs=pl.BlockSpec((_TC_BLK, d), lambda i: (i, 0)),
        out_shape=jax.ShapeDtypeStruct((n, d), jnp.float32),
    )(idx3, table)


_SC_FRAC_NUM, _SC_FRAC_DEN = 1, 2


def kernel(input, table):
    d = table.shape[-1]
    idx = input.reshape(-1).astype(jnp.int32)
    tablef = table.astype(jnp.float32)
    n = idx.shape[0]
    grain = 32 * _CHUNK
    n_sc = (n * _SC_FRAC_NUM // _SC_FRAC_DEN) // grain * grain
    out_sc = _lookup(idx[:n_sc], tablef)
    out_tc = _tc_lookup(idx[n_sc:], tablef)
    out = jnp.concatenate([out_sc, out_tc], axis=0)
    return out.reshape(input.shape + (d,))


# speedup vs baseline: 1.2300x; 1.2300x over previous
import functools

import jax
import jax.numpy as jnp
from jax import lax
from jax.experimental import pallas as pl
from jax.experimental.pallas import tpu as pltpu
from jax.experimental.pallas import tpu_sc as plsc

_CHUNK = 128
_NBUF = 4
_TC_BLK = 2048
_SC_FRAC_NUM, _SC_FRAC_DEN = 1, 2


def _sc_head(idx, table, out_ref, n_sc):
    v, d = table.shape
    chunk = _CHUNK
    info = plsc.get_sparse_core_info()
    nw = info.num_cores * info.num_subcores
    b_per_w = n_sc // nw
    n_chunks = b_per_w // chunk
    nbuf = min(_NBUF, n_chunks)

    mesh = plsc.VectorSubcoreMesh(core_axis_name="c", subcore_axis_name="s")

    @functools.partial(
        pl.kernel,
        mesh=mesh,
        scratch_types=(
            [pltpu.VMEM_SHARED((v, d), jnp.float32)]
            + [pltpu.VMEM((b_per_w,), jnp.int32)]
            + [pltpu.VMEM((nbuf * chunk, d), jnp.float32)]
            + [pltpu.SemaphoreType.DMA((nbuf,))]
            + [pltpu.SemaphoreType.DMA((nbuf,))]
        ),
    )
    def k(table_hbm, idx_hbm, out_hbm, table_sh, idx_v, buf, gsems, ssems):
        sid = lax.axis_index("s")
        wid = sid * info.num_cores + lax.axis_index("c")
        @pl.when(sid == 0)
        def _():
            pltpu.sync_copy(table_hbm, table_sh)

        pltpu.sync_copy(idx_hbm.at[pl.ds(wid * b_per_w, b_per_w)], idx_v)
        plsc.subcore_barrier()

        out_base = wid * b_per_w

        def start_gather(c, slot, b):
            pltpu.async_copy(
                table_sh.at[idx_v.at[pl.ds(c * chunk, chunk)]],
                buf.at[pl.ds(slot, chunk)],
                gsems.at[b],
            )

        for b in range(nbuf):
            start_gather(b, b * chunk, b)

        def chunk_body(c, _):
            b = lax.rem(c, nbuf)
            slot = b * chunk
            bufslot = buf.at[pl.ds(slot, chunk)]
            pltpu.make_async_copy(
                out_hbm.at[pl.ds(out_base, chunk)], bufslot, gsems.at[b]
            ).wait()
            pltpu.async_copy(
                bufslot,
                out_hbm.at[pl.ds(out_base + c * chunk, chunk)],
                ssems.at[b],
            )

            @pl.when(c < n_chunks - nbuf)
            def _():
                pltpu.make_async_copy(
                    bufslot, out_hbm.at[pl.ds(out_base, chunk)], ssems.at[b]
                ).wait()
                start_gather(c + nbuf, slot, b)

            return 0

        lax.fori_loop(0, n_chunks, chunk_body, 0)
        for b in range(nbuf):
            pltpu.make_async_copy(
                buf.at[pl.ds(b * chunk, chunk)],
                out_hbm.at[pl.ds(out_base, chunk)],
                ssems.at[b],
            ).wait()

    k(table, idx, out_ref)


def _tc_tail(idx, table, n_sc):
    (n,) = idx.shape
    v, d = table.shape
    nb_tail = (n - n_sc) // _TC_BLK
    hb = n_sc // _TC_BLK
    idx3 = idx.reshape(n // _TC_BLK, 1, _TC_BLK)

    def body(idx_ref, table_ref, out_ref):
        idxb = idx_ref[0, 0, :]
        iota = lax.broadcasted_iota(jnp.int32, (_TC_BLK, v), 1)
        oh = (idxb[:, None] == iota).astype(jnp.float32)
        out_ref[...] = jnp.dot(
            oh, table_ref[...], preferred_element_type=jnp.float32
        )

    return pl.pallas_call(
        body,
        grid=(nb_tail,),
        in_specs=[
            pl.BlockSpec((1, 1, _TC_BLK), lambda i, hb=hb: (i + hb, 0, 0)),
            pl.BlockSpec((v, d), lambda i: (0, 0)),
        ],
        out_specs=pl.BlockSpec((_TC_BLK, d), lambda i, hb=hb: (i + hb, 0)),
        out_shape=jax.ShapeDtypeStruct((n, d), jnp.float32),
    )(idx3, table)


def kernel(input, table):
    d = table.shape[-1]
    idx = input.reshape(-1).astype(jnp.int32)
    tablef = table.astype(jnp.float32)
    n = idx.shape[0]
    grain = 32 * _CHUNK
    n_sc = (n * _SC_FRAC_NUM // _SC_FRAC_DEN) // grain * grain
    out = _tc_tail(idx, tablef, n_sc)
    ref = jax.new_ref(out)
    _sc_head(idx, tablef, ref, n_sc)
    return jax.freeze(ref).reshape(input.shape + (d,))


# speedup vs baseline: 1.4373x; 1.1685x over previous
import functools

import jax
import jax.numpy as jnp
from jax import lax
from jax.experimental import pallas as pl
from jax.experimental.pallas import tpu as pltpu
from jax.experimental.pallas import tpu_sc as plsc

_CHUNK = 128
_NBUF = 4


def _lookup(idx, table):
    (n_rows,) = idx.shape
    v, d = table.shape
    chunk = _CHUNK
    info = plsc.get_sparse_core_info()
    nw = info.num_cores * info.num_subcores
    b_per_w = n_rows // nw
    n_chunks = b_per_w // chunk
    nbuf = min(_NBUF, n_chunks)

    mesh = plsc.VectorSubcoreMesh(core_axis_name="c", subcore_axis_name="s")

    @functools.partial(
        pl.kernel,
        mesh=mesh,
        out_type=jax.ShapeDtypeStruct((n_rows, d), jnp.float32),
        scratch_types=(
            [pltpu.VMEM_SHARED((v, d), jnp.float32)]
            + [pltpu.VMEM((b_per_w,), jnp.int32)]
            + [pltpu.VMEM((nbuf * chunk, d), jnp.float32)]
            + [pltpu.SemaphoreType.DMA((nbuf,))]
            + [pltpu.SemaphoreType.DMA((nbuf,))]
        ),
    )
    def k(table_hbm, idx_hbm, out_hbm, table_sh, idx_v, buf, gsems, ssems):
        sid = lax.axis_index("s")
        wid = sid * info.num_cores + lax.axis_index("c")
        @pl.when(sid == 0)
        def _():
            pltpu.sync_copy(table_hbm, table_sh)

        pltpu.sync_copy(idx_hbm.at[pl.ds(wid * b_per_w, b_per_w)], idx_v)
        plsc.subcore_barrier()

        out_base = wid * b_per_w

        def start_gather(c, slot, b):
            pltpu.async_copy(
                table_sh.at[idx_v.at[pl.ds(c * chunk, chunk)]],
                buf.at[pl.ds(slot, chunk)],
                gsems.at[b],
            )

        for b in range(nbuf):
            start_gather(b, b * chunk, b)

        def chunk_body(c, _):
            b = lax.rem(c, nbuf)
            slot = b * chunk
            bufslot = buf.at[pl.ds(slot, chunk)]
            pltpu.make_async_copy(
                out_hbm.at[pl.ds(out_base, chunk)], bufslot, gsems.at[b]
            ).wait()
            pltpu.async_copy(
                bufslot,
                out_hbm.at[pl.ds(out_base + c * chunk, chunk)],
                ssems.at[b],
            )

            @pl.when(c < n_chunks - nbuf)
            def _():
                pltpu.make_async_copy(
                    bufslot, out_hbm.at[pl.ds(out_base, chunk)], ssems.at[b]
                ).wait()
                start_gather(c + nbuf, slot, b)

            return 0

        lax.fori_loop(0, n_chunks, chunk_body, 0)
        for b in range(nbuf):
            pltpu.make_async_copy(
                buf.at[pl.ds(b * chunk, chunk)],
                out_hbm.at[pl.ds(out_base, chunk)],
                ssems.at[b],
            ).wait()

    return k(table, idx)


def kernel(input, table):
    d = table.shape[-1]
    idx = input.reshape(-1).astype(jnp.int32)
    out = _lookup(idx, table.astype(jnp.float32))
    return out.reshape(input.shape + (d,))
